# Initial kernel scaffold; baseline (speedup 1.0000x reference)
import jax
import jax.numpy as jnp
from jax.experimental import pallas as pl


def kernel(x, edge_index, edge_weight, W1, b1, g1, be1, W2, b2, g2, be2):
    raise NotImplementedError("write your pallas kernel here")


# speedup vs baseline: 8.3674x; 8.3674x over previous
import functools

import jax
import jax.numpy as jnp
from jax import lax
from jax.experimental import pallas as pl
from jax.experimental.pallas import tpu as pltpu
from jax.experimental.pallas import tpu_sc as plsc

_N = 10000
_E = 320000
_D = 128
_EPS = 1e-5

_NC = 2
_NS = 16
_NW = _NC * _NS
_EPT = _E // _NW
_CH = 80
_NCHUNK = _EPT // _CH
_NPAD = 10240
_RPT = _NPAD // _NS
_ZR = 128

@functools.cache
def _sc_kernels():
    mesh = plsc.VectorSubcoreMesh(core_axis_name="c", subcore_axis_name="s",
                                  num_cores=_NC, num_subcores=_NS)
    deg = _make_deg_kernel(mesh)
    msg = _make_msg_kernel(mesh)
    return deg, msg


def _make_deg_kernel(mesh):
    return functools.partial(
        pl.kernel,
        out_type=jax.ShapeDtypeStruct((_NC, _NPAD, _D), jnp.float32),
        mesh=mesh,
        scratch_types=[
            pltpu.VMEM((_CH,), jnp.int32),
            pltpu.VMEM((_CH,), jnp.float32),
            pltpu.VMEM((_CH, _D), jnp.float32),
            pltpu.VMEM((_ZR, _D), jnp.float32),
            pltpu.VMEM_SHARED((_NPAD, _D), jnp.float32),
        ],
    )(_deg_body)


def _deg_body(dst_hbm, ew_hbm, out_hbm, didx, ewb, val2d, zbuf, deg_sh):
    cid = lax.axis_index("c")
    sid = lax.axis_index("s")
    zvec = jnp.zeros((16,), jnp.float32)

    def zrow(i, _):
        for f in range(_D // 16):
            zbuf[i, pl.ds(f * 16, 16)] = zvec
        return 0

    lax.fori_loop(0, _ZR, zrow, 0)
    for j in range(_RPT // _ZR):
        pltpu.sync_copy(zbuf, deg_sh.at[pl.ds(sid * _RPT + j * _ZR, _ZR)])
    plsc.subcore_barrier()

    ebase = (cid * _NS + sid) * _EPT

    def chunk(i, _):
        base = ebase + i * _CH
        pltpu.sync_copy(dst_hbm.at[pl.ds(base, _CH)], didx)
        pltpu.sync_copy(ew_hbm.at[pl.ds(base, _CH)], ewb)
        for g in range(_CH // 16):
            wv = ewb[pl.ds(g * 16, 16)]
            for k in range(16):
                row = jnp.full((16,), wv[k], jnp.float32)
                for f in range(_D // 16):
                    val2d[g * 16 + k, pl.ds(f * 16, 16)] = row
        pltpu.sync_copy(val2d, deg_sh.at[didx], add=True)
        return 0

    lax.fori_loop(0, _NCHUNK, chunk, 0)
    plsc.subcore_barrier()
    pltpu.sync_copy(
        deg_sh.at[pl.ds(sid * _RPT, _RPT)],
        out_hbm.at[cid, pl.ds(sid * _RPT, _RPT)],
    )


def _make_msg_kernel(mesh):
    return functools.partial(
        pl.kernel,
        out_type=jax.ShapeDtypeStruct((_NC, _NPAD, _D), jnp.float32),
        mesh=mesh,
        scratch_types=[
            pltpu.VMEM((_CH,), jnp.int32),
            pltpu.VMEM((_CH,), jnp.int32),
            pltpu.VMEM((_CH,), jnp.float32),
            pltpu.VMEM((_CH, _D), jnp.float32),
            pltpu.VMEM((_ZR, _D), jnp.float32),
            pltpu.VMEM_SHARED((_NPAD, _D), jnp.float32),
            pltpu.SemaphoreType.DMA,
        ],
    )(_msg_body)


def _msg_body(src_hbm, dst_hbm, ew_hbm, hp_hbm, out_hbm,
              sidx, didx, ewb, rows, zbuf, acc_sh, sem):
    cid = lax.axis_index("c")
    sid = lax.axis_index("s")
    zvec = jnp.zeros((16,), jnp.float32)

    def zrow(i, _):
        for f in range(_D // 16):
            zbuf[i, pl.ds(f * 16, 16)] = zvec
        return 0

    lax.fori_loop(0, _ZR, zrow, 0)
    for j in range(_RPT // _ZR):
        pltpu.sync_copy(zbuf, acc_sh.at[pl.ds(sid * _RPT + j * _ZR, _ZR)])
    plsc.subcore_barrier()

    ebase = (cid * _NS + sid) * _EPT

    def chunk(i, _):
        base = ebase + i * _CH
        pltpu.sync_copy(src_hbm.at[pl.ds(base, _CH)], sidx)
        pltpu.sync_copy(dst_hbm.at[pl.ds(base, _CH)], didx)
        pltpu.sync_copy(ew_hbm.at[pl.ds(base, _CH)], ewb)
        pltpu.async_copy(hp_hbm.at[sidx], rows, sem).wait()
        for g in range(_CH // 16):
            wv = ewb[pl.ds(g * 16, 16)]
            for k in range(16):
                e = g * 16 + k
                w = wv[k]
                for f in range(_D // 16):
                    rows[e, pl.ds(f * 16, 16)] = rows[e, pl.ds(f * 16, 16)] * w
        pltpu.sync_copy(rows, acc_sh.at[didx], add=True)
        return 0

    lax.fori_loop(0, _NCHUNK, chunk, 0)
    plsc.subcore_barrier()
    pltpu.sync_copy(
        acc_sh.at[pl.ds(sid * _RPT, _RPT)],
        out_hbm.at[cid, pl.ds(sid * _RPT, _RPT)],
    )


def _tc_a_body(x_ref, w1_ref, degp_ref, h1p_ref, dis_ref):
    deg = (degp_ref[0, :_N, 0:1] + degp_ref[1, :_N, 0:1]) + 1.0
    dis = lax.rsqrt(deg)
    h = lax.dot_general(x_ref[...], w1_ref[...],
                        (((1,), (1,)), ((), ())),
                        preferred_element_type=jnp.float32)
    h1p_ref[...] = h * dis
    dis_ref[...] = dis


def _tc_a(x, W1, degp):
    return pl.pallas_call(
        _tc_a_body,
        out_shape=(jax.ShapeDtypeStruct((_N, _D), jnp.float32),
                   jax.ShapeDtypeStruct((_N, 1), jnp.float32)),
    )(x, W1, degp)


def _bn(v, gamma, beta):
    mean = jnp.mean(v, axis=0, keepdims=True)
    var = jnp.mean((v - mean) ** 2, axis=0, keepdims=True)
    return gamma * ((v - mean) * lax.rsqrt(var + _EPS)) + beta


def _tc_b_body(s_ref, h1p_ref, dis_ref, b1_ref, g1_ref, be1_ref, w2_ref,
               h2p_ref):
    conv = (dis_ref[...] * (s_ref[0, :_N, :] + s_ref[1, :_N, :] + h1p_ref[...])
            + b1_ref[...])
    x1 = jax.nn.gelu(_bn(conv, g1_ref[...], be1_ref[...]))
    h2 = lax.dot_general(x1, w2_ref[...],
                         (((1,), (1,)), ((), ())),
                         preferred_element_type=jnp.float32)
    h2p_ref[...] = h2 * dis_ref[...]


def _tc_b(S1, h1p, dis, b1, g1, be1, W2):
    return pl.pallas_call(
        _tc_b_body,
        out_shape=jax.ShapeDtypeStruct((_N, _D), jnp.float32),
    )(S1, h1p, dis, b1, g1, be1, W2)


def _tc_c_body(s_ref, h2p_ref, dis_ref, b2_ref, g2_ref, be2_ref, x_ref,
               out_ref):
    conv = (dis_ref[...] * (s_ref[0, :_N, :] + s_ref[1, :_N, :] + h2p_ref[...])
            + b2_ref[...])
    out_ref[...] = _bn(conv, g2_ref[...], be2_ref[...]) + x_ref[...]


def _tc_c(S2, h2p, dis, b2, g2, be2, x):
    return pl.pallas_call(
        _tc_c_body,
        out_shape=jax.ShapeDtypeStruct((_N, _D), jnp.float32),
    )(S2, h2p, dis, b2, g2, be2, x)


def kernel(x, edge_index, edge_weight, W1, b1, g1, be1, W2, b2, g2, be2):
    src = edge_index[0]
    dst = edge_index[1]
    b1r = b1.reshape(1, _D)
    g1r = g1.reshape(1, _D)
    be1r = be1.reshape(1, _D)
    b2r = b2.reshape(1, _D)
    g2r = g2.reshape(1, _D)
    be2r = be2.reshape(1, _D)

    deg_kernel, msg_kernel = _sc_kernels()
    degp = deg_kernel(dst, edge_weight)
    h1p, dis = _tc_a(x, W1, degp)
    S1 = msg_kernel(src, dst, edge_weight, h1p)
    h2p = _tc_b(S1, h1p, dis, b1r, g1r, be1r, W2)
    S2 = msg_kernel(src, dst, edge_weight, h2p)
    return _tc_c(S2, h2p, dis, b2r, g2r, be2r, x)


# speedup vs baseline: 18.2111x; 2.1764x over previous
import functools

import jax
import jax.numpy as jnp
from jax import lax
from jax.experimental import pallas as pl
from jax.experimental.pallas import tpu as pltpu
from jax.experimental.pallas import tpu_sc as plsc

_N = 10000
_E = 320000
_D = 128
_DH = _D // 2
_EPS = 1e-5

_NC = 2
_NS = 16
_NW = _NC * _NS
_CH = 80

_EPTD = _E // _NW
_NCHD = _EPTD // _CH
_EPTM = _E // _NS
_NCHM = _EPTM // _CH

_NPAD = 10240
_RPT = _NPAD // _NS
_ZR = 128


@functools.cache
def _sc_kernels():
    mesh = plsc.VectorSubcoreMesh(core_axis_name="c", subcore_axis_name="s",
                                  num_cores=_NC, num_subcores=_NS)
    deg = _make_deg_kernel(mesh)
    msg = _make_msg_kernel(mesh)
    return deg, msg


def _make_deg_kernel(mesh):
    return functools.partial(
        pl.kernel,
        out_type=jax.ShapeDtypeStruct((_NC, _NPAD, 16), jnp.float32),
        mesh=mesh,
        scratch_types=[
            pltpu.VMEM((_NCHD, _CH), jnp.int32),
            pltpu.VMEM((_NCHD, _CH), jnp.float32),
            pltpu.VMEM((_CH, 16), jnp.float32),
            pltpu.VMEM((_ZR, 16), jnp.float32),
            pltpu.VMEM_SHARED((_NPAD, 16), jnp.float32),
        ],
        compiler_params=pltpu.CompilerParams(use_tc_tiling_on_sc=False),
    )(_deg_body)


def _deg_body(dst_hbm, ew_hbm, out_hbm, didx2, ewb2, val2d, zbuf, deg_sh):
    cid = lax.axis_index("c")
    sid = lax.axis_index("s")
    wid = cid * _NS + sid
    zvec = jnp.zeros((16,), jnp.float32)

    def zrow(i, _):
        zbuf[i, :] = zvec
        return 0

    lax.fori_loop(0, _ZR, zrow, 0)
    for j in range(_RPT // _ZR):
        pltpu.sync_copy(zbuf, deg_sh.at[pl.ds(sid * _RPT + j * _ZR, _ZR)])

    pltpu.sync_copy(dst_hbm.at[wid], didx2)
    pltpu.sync_copy(ew_hbm.at[wid], ewb2)
    plsc.subcore_barrier()

    def chunk(c, _):
        for g in range(_CH // 16):
            wv = ewb2[c, pl.ds(g * 16, 16)]
            for k in range(16):
                val2d[g * 16 + k, :] = jnp.full((16,), wv[k], jnp.float32)
        pltpu.sync_copy(val2d, deg_sh.at[didx2.at[c]], add=True)
        return 0

    lax.fori_loop(0, _NCHD, chunk, 0)
    plsc.subcore_barrier()
    pltpu.sync_copy(
        deg_sh.at[pl.ds(sid * _RPT, _RPT)],
        out_hbm.at[cid, pl.ds(sid * _RPT, _RPT)],
    )


def _make_msg_kernel(mesh):
    return functools.partial(
        pl.kernel,
        out_type=jax.ShapeDtypeStruct((_NC, _NPAD, _DH), jnp.float32),
        mesh=mesh,
        scratch_types=[
            pltpu.VMEM((_NCHM, _CH), jnp.int32),
            pltpu.VMEM((_NCHM, _CH), jnp.int32),
            pltpu.VMEM((_NCHM, _CH), jnp.float32),
            pltpu.VMEM((_CH, _DH), jnp.float32),
            pltpu.VMEM((_CH, _DH), jnp.float32),
            pltpu.VMEM((_ZR, _DH), jnp.float32),
            pltpu.VMEM_SHARED((_NPAD, _DH), jnp.float32),
            pltpu.SemaphoreType.DMA,
            pltpu.SemaphoreType.DMA,
        ],
        compiler_params=pltpu.CompilerParams(use_tc_tiling_on_sc=False),
    )(_msg_body)


def _msg_body(src_hbm, dst_hbm, ew_hbm, hp_hbm, out_hbm,
              sidx2, didx2, ewb2, rows0, rows1, zbuf, acc_sh, gsem0, gsem1):
    cid = lax.axis_index("c")
    sid = lax.axis_index("s")
    zvec = jnp.zeros((16,), jnp.float32)

    def zrow(i, _):
        for f in range(_DH // 16):
            zbuf[i, pl.ds(f * 16, 16)] = zvec
        return 0

    lax.fori_loop(0, _ZR, zrow, 0)
    for j in range(_RPT // _ZR):
        pltpu.sync_copy(zbuf, acc_sh.at[pl.ds(sid * _RPT + j * _ZR, _ZR)])

    pltpu.sync_copy(src_hbm.at[sid], sidx2)
    pltpu.sync_copy(dst_hbm.at[sid], didx2)
    pltpu.sync_copy(ew_hbm.at[sid], ewb2)
    plsc.subcore_barrier()

    rows = (rows0, rows1)
    gsem = (gsem0, gsem1)
    table = hp_hbm.at[cid]

    pltpu.async_copy(table.at[sidx2.at[0]], rows0, gsem0)

    def substep(c, b):
        @pl.when(c + 1 < _NCHM)
        def _():
            pltpu.async_copy(table.at[sidx2.at[c + 1]], rows[1 - b],
                             gsem[1 - b])
        pltpu.make_async_copy(table.at[sidx2.at[c]], rows[b], gsem[b]).wait()
        for g in range(_CH // 16):
            wv = ewb2[c, pl.ds(g * 16, 16)]
            for k in range(16):
                e = g * 16 + k
                w = wv[k]
                for f in range(_DH // 16):
                    rows[b][e, pl.ds(f * 16, 16)] = (
                        rows[b][e, pl.ds(f * 16, 16)] * w)
        pltpu.sync_copy(rows[b], acc_sh.at[didx2.at[c]], add=True)

    def pair(j, _):
        substep(2 * j, 0)
        substep(2 * j + 1, 1)
        return 0

    lax.fori_loop(0, _NCHM // 2, pair, 0)

    plsc.subcore_barrier()
    pltpu.sync_copy(
        acc_sh.at[pl.ds(sid * _RPT, _RPT)],
        out_hbm.at[cid, pl.ds(sid * _RPT, _RPT)],
    )


def _tc_a_body(x_ref, w1_ref, degp_ref, h1p_ref, dis_ref):
    deg = (degp_ref[0, :_N, 0:1] + degp_ref[1, :_N, 0:1]) + 1.0
    dis = lax.rsqrt(deg)
    h = lax.dot_general(x_ref[...], w1_ref[...],
                        (((1,), (1,)), ((), ())),
                        preferred_element_type=jnp.float32)
    hd = h * dis
    h1p_ref[0] = hd[:, :_DH]
    h1p_ref[1] = hd[:, _DH:]
    dis_ref[...] = dis


def _tc_a(x, W1, degp):
    return pl.pallas_call(
        _tc_a_body,
        out_shape=(jax.ShapeDtypeStruct((_NC, _N, _DH), jnp.float32),
                   jax.ShapeDtypeStruct((_N, 1), jnp.float32)),
    )(x, W1, degp)


def _bn(v, gamma, beta):
    mean = jnp.mean(v, axis=0, keepdims=True)
    var = jnp.mean((v - mean) ** 2, axis=0, keepdims=True)
    return gamma * ((v - mean) * lax.rsqrt(var + _EPS)) + beta


def _tc_b_body(s_ref, h1p_ref, dis_ref, b1_ref, g1_ref, be1_ref, w2_ref,
               h2p_ref):
    s_full = jnp.concatenate([s_ref[0, :_N, :], s_ref[1, :_N, :]], axis=1)
    h_full = jnp.concatenate([h1p_ref[0], h1p_ref[1]], axis=1)
    conv = dis_ref[...] * (s_full + h_full) + b1_ref[...]
    x1 = jax.nn.gelu(_bn(conv, g1_ref[...], be1_ref[...]))
    h2 = lax.dot_general(x1, w2_ref[...],
                         (((1,), (1,)), ((), ())),
                         preferred_element_type=jnp.float32)
    hd = h2 * dis_ref[...]
    h2p_ref[0] = hd[:, :_DH]
    h2p_ref[1] = hd[:, _DH:]


def _tc_b(S1, h1p, dis, b1, g1, be1, W2):
    return pl.pallas_call(
        _tc_b_body,
        out_shape=jax.ShapeDtypeStruct((_NC, _N, _DH), jnp.float32),
    )(S1, h1p, dis, b1, g1, be1, W2)


def _tc_c_body(s_ref, h2p_ref, dis_ref, b2_ref, g2_ref, be2_ref, x_ref,
               out_ref):
    s_full = jnp.concatenate([s_ref[0, :_N, :], s_ref[1, :_N, :]], axis=1)
    h_full = jnp.concatenate([h2p_ref[0], h2p_ref[1]], axis=1)
    conv = dis_ref[...] * (s_full + h_full) + b2_ref[...]
    out_ref[...] = _bn(conv, g2_ref[...], be2_ref[...]) + x_ref[...]


def _tc_c(S2, h2p, dis, b2, g2, be2, x):
    return pl.pallas_call(
        _tc_c_body,
        out_shape=jax.ShapeDtypeStruct((_N, _D), jnp.float32),
    )(S2, h2p, dis, b2, g2, be2, x)


def kernel(x, edge_index, edge_weight, W1, b1, g1, be1, W2, b2, g2, be2):
    src_m = edge_index[0].reshape(_NS, _NCHM, _CH)
    dst_m = edge_index[1].reshape(_NS, _NCHM, _CH)
    ew_m = edge_weight.reshape(_NS, _NCHM, _CH)
    dst_d = edge_index[1].reshape(_NW, _NCHD, _CH)
    ew_d = edge_weight.reshape(_NW, _NCHD, _CH)
    b1r = b1.reshape(1, _D)
    g1r = g1.reshape(1, _D)
    be1r = be1.reshape(1, _D)
    b2r = b2.reshape(1, _D)
    g2r = g2.reshape(1, _D)
    be2r = be2.reshape(1, _D)

    deg_kernel, msg_kernel = _sc_kernels()
    degp = deg_kernel(dst_d, ew_d)
    h1p, dis = _tc_a(x, W1, degp)
    S1 = msg_kernel(src_m, dst_m, ew_m, h1p)
    h2p = _tc_b(S1, h1p, dis, b1r, g1r, be1r, W2)
    S2 = msg_kernel(src_m, dst_m, ew_m, h2p)
    return _tc_c(S2, h2p, dis, b2r, g2r, be2r, x)
